# Initial kernel scaffold; baseline (speedup 1.0000x reference)
import jax
import jax.numpy as jnp
from jax.experimental import pallas as pl


def kernel(logits, topK, topP, q, eps, isNeedLogits, topKGuess):
    raise NotImplementedError("write your pallas kernel here")


# speedup vs baseline: 7.4925x; 7.4925x over previous
import functools

import jax
import jax.numpy as jnp
import numpy as np
from jax import lax
from jax.experimental import pallas as pl
from jax.experimental.pallas import tpu as pltpu
from jax.experimental.pallas import tpu_sc as plsc

BATCH = 128
VOCAB = 100000
L = 16
NC, NS = 2, 16
NW = NC * NS
ROWS_PER_W = BATCH // NW
CHUNKS = VOCAB // L
UNROLL = 10
NITER = CHUNKS // UNROLL
NEG_INF = np.float32(-np.inf)
MASK31 = np.int32(0x7FFFFFFF)
ONE_BITS_P1 = np.int32(0x3F800001)
BIG_I32 = np.int32(0x7FFFFFFF)


def _keys(x):
    b = plsc.bitcast(x, jnp.int32)
    return jnp.where(b < 0, b ^ MASK31, b)


def _unkey_scalar(k):
    kv = jnp.full((L,), k, jnp.int32)
    b = jnp.where(kv < 0, kv ^ MASK31, kv)
    return jnp.max(plsc.bitcast(b, jnp.float32))


def _bits_to_f32_scalar(k):
    kv = jnp.full((L,), k, jnp.int32)
    return jnp.max(plsc.bitcast(kv, jnp.float32))


def _mid(lo, hi):
    return (lo >> 1) + (hi >> 1) + (lo & hi & 1)


def _sc_body(logits_hbm, topk_hbm, topp_hbm, out_l_hbm, out_idx_hbm,
             row_v, topk_v, topp_v, idx_v):
    wid = lax.axis_index("s") * NC + lax.axis_index("c")
    pltpu.sync_copy(topk_hbm, topk_v)
    pltpu.sync_copy(topp_hbm, topp_v)
    iota = lax.iota(jnp.int32, L)

    idx_vec = jnp.zeros((L,), jnp.int32)
    for j in range(ROWS_PER_W):
        row = wid * ROWS_PER_W + j
        pltpu.sync_copy(logits_hbm.at[row], row_v)
        base = (row // L) * L
        lane = row - base
        kvec = topk_v[pl.ds(base, L)]
        pvec = topp_v[pl.ds(base, L)]
        k = jnp.sum(jnp.where(iota == lane, kvec, jnp.int32(0)))
        p = jnp.sum(jnp.where(iota == lane, pvec, jnp.float32(0.0)))

        def p0_body(i, carry):
            kmaxv, kminv, fmaxv = carry
            for u in range(UNROLL):
                x = row_v[pl.ds((i * UNROLL + u) * L, L)]
                ky = _keys(x)
                kmaxv = jnp.maximum(kmaxv, ky)
                kminv = jnp.minimum(kminv, ky)
                fmaxv = jnp.maximum(fmaxv, x)
            return kmaxv, kminv, fmaxv

        kmaxv, kminv, fmaxv = lax.fori_loop(
            0, NITER, p0_body,
            (jnp.full((L,), jnp.int32(-0x80000000)),
             jnp.full((L,), BIG_I32),
             jnp.full((L,), NEG_INF)))
        kmax = jnp.max(kmaxv)
        kmin = jnp.min(kminv)
        m = jnp.max(fmaxv)

        kk = jnp.maximum(k, 1)

        def bk_body(_, carry):
            lo, hi = carry
            mid = _mid(lo, hi)

            def cnt_body(i, acc):
                for u in range(UNROLL):
                    x = row_v[pl.ds((i * UNROLL + u) * L, L)]
                    acc = acc + jnp.where(_keys(x) >= mid,
                                          jnp.int32(1), jnp.int32(0))
                return acc

            acc = lax.fori_loop(0, NITER, cnt_body, jnp.zeros((L,), jnp.int32))
            ok = jnp.sum(acc) >= kk
            return jnp.where(ok, mid, lo), jnp.where(ok, hi, mid)

        lo_k, _ = lax.fori_loop(0, 32, bk_body, (kmin, kmax + 1))
        vk = _unkey_scalar(lo_k)
        apply_k = jnp.logical_and(k >= 1, k <= 1024)
        thresh = jnp.where(apply_k, vk, NEG_INF)

        def ps_body(i, carry):
            accS, accidx = carry
            for u in range(UNROLL):
                c = i * UNROLL + u
                x = row_v[pl.ds(c * L, L)]
                e = jnp.where(x < thresh, jnp.float32(0.0), jnp.exp(x - m))
                accS = accS + e
                gidx = iota + c * L
                accidx = jnp.minimum(accidx, jnp.where(x == m, gidx, BIG_I32))
            return accS, accidx

        accS, accidx = lax.fori_loop(
            0, NITER, ps_body,
            (jnp.zeros((L,), jnp.float32), jnp.full((L,), BIG_I32)))
        S = jnp.sum(accS)
        best_idx = jnp.min(accidx)
        Te = p * S

        def bp_body(_, carry):
            lo, hi = carry
            mid = _mid(lo, hi)

            def mass_body(i, acc):
                for u in range(UNROLL):
                    x = row_v[pl.ds((i * UNROLL + u) * L, L)]
                    e = jnp.where(x < thresh, jnp.float32(0.0), jnp.exp(x - m))
                    eb = plsc.bitcast(e, jnp.int32)
                    acc = acc + jnp.where(eb >= mid, e, jnp.float32(0.0))
                return acc

            acc = lax.fori_loop(0, NITER, mass_body, jnp.zeros((L,), jnp.float32))
            mass = jnp.sum(acc)
            ok = jnp.logical_and(mass >= Te, mass > 0)
            return jnp.where(ok, mid, lo), jnp.where(ok, hi, mid)

        lo_p, _ = lax.fori_loop(0, 31, bp_body,
                                (jnp.int32(0), ONE_BITS_P1))
        estar = _bits_to_f32_scalar(lo_p)

        def pf_body(i, _):
            for u in range(UNROLL):
                c = i * UNROLL + u
                x = row_v[pl.ds(c * L, L)]
                e = jnp.where(x < thresh, jnp.float32(0.0), jnp.exp(x - m))
                drop = jnp.logical_or(x < thresh, e < estar)
                row_v[pl.ds(c * L, L)] = jnp.where(drop, NEG_INF, x)
            return 0

        lax.fori_loop(0, NITER, pf_body, 0)
        pltpu.sync_copy(row_v, out_l_hbm.at[row])
        idx_vec = jnp.where(iota == j, jnp.full((L,), best_idx), idx_vec)

    idx_v[...] = idx_vec
    pltpu.sync_copy(idx_v, out_idx_hbm.at[wid])


@jax.jit
def _run(logits, topK, topP):
    mesh = plsc.VectorSubcoreMesh(core_axis_name="c", subcore_axis_name="s",
                                  num_cores=NC, num_subcores=NS)
    out_l, out_idx = pl.kernel(
        _sc_body,
        out_type=[
            jax.ShapeDtypeStruct((BATCH, VOCAB), jnp.float32),
            jax.ShapeDtypeStruct((NW, L), jnp.int32),
        ],
        mesh=mesh,
        scratch_types=[
            pltpu.VMEM((VOCAB,), jnp.float32),
            pltpu.VMEM((BATCH,), jnp.int32),
            pltpu.VMEM((BATCH,), jnp.float32),
            pltpu.VMEM((L,), jnp.int32),
        ],
        compiler_params=pltpu.CompilerParams(needs_layout_passes=False),
    )(logits, topK, topP)
    return out_idx[:, :ROWS_PER_W].reshape(BATCH), out_l


def kernel(logits, topK, topP, q, eps, isNeedLogits, topKGuess):
    idx, out_l = _run(logits.astype(jnp.float32), topK.astype(jnp.int32),
                      topP.astype(jnp.float32))
    return idx, out_l
